# Initial kernel scaffold; baseline (speedup 1.0000x reference)
import functools

import jax
import jax.numpy as jnp
from jax import lax
from jax.experimental import pallas as pl
from jax.experimental.pallas import tpu as pltpu
from jax.experimental.pallas import tpu_sc as plsc

TEMPERATURE = 0.1
N_NEG = 10
EPS = 1e-8

B, D, H, W = 8, 512, 8, 128
T = H * W
LANES = 16
NW = 32
PER_W = (B * T * LANES) // NW
NJ = PER_W // 128
FIRE = 4


def _gram_body(o_ref, p_ref, out_ref):
    o = o_ref[0]
    p = p_ref[0]
    dots = lax.dot_general(o, p, (((1,), (1,)), ((), ())),
                           preferred_element_type=jnp.float32)
    no = jnp.maximum(jnp.sqrt(jnp.sum(o * o, axis=1, keepdims=True)), EPS)
    npv = jnp.maximum(jnp.sqrt(jnp.sum(p * p, axis=1)), EPS).reshape(1, T)
    out_ref[0] = dots * (1.0 / no) * (1.0 / npv) * (1.0 / TEMPERATURE)


def _gram(orig_t, pred_z):
    return pl.pallas_call(
        _gram_body,
        grid=(B,),
        in_specs=[
            pl.BlockSpec((1, T, D), lambda b: (b, 0, 0)),
            pl.BlockSpec((1, T, D), lambda b: (b, 0, 0)),
        ],
        out_specs=pl.BlockSpec((1, T, T), lambda b: (b, 0, 0)),
        out_shape=jax.ShapeDtypeStruct((B, T, T), jnp.float32),
    )(orig_t, pred_z)


@functools.partial(
    pl.kernel,
    mesh=plsc.VectorSubcoreMesh(core_axis_name="c", subcore_axis_name="s"),
    out_type=jax.ShapeDtypeStruct((NW, NJ, 128, 1), jnp.float32),
    scratch_types=[
        pltpu.VMEM((NJ, 128), jnp.int32),
        pltpu.VMEM((NJ, 128, 1), jnp.float32),
        pltpu.SemaphoreType.DMA,
    ],
)
def _sc_gather(table_hbm, idx_hbm, out_hbm, idx_v, rows_v, sem):
    wid = lax.axis_index("s") * 2 + lax.axis_index("c")
    pltpu.sync_copy(idx_hbm.at[wid], idx_v)

    def group(g, carry):
        copies = [
            pltpu.async_copy(table_hbm.at[idx_v.at[g * FIRE + k]],
                             rows_v.at[g * FIRE + k], sem)
            for k in range(FIRE)
        ]
        for c in copies:
            c.wait()
        return carry

    lax.fori_loop(0, NJ // FIRE, group, 0)
    pltpu.sync_copy(rows_v, out_hbm.at[wid])


def _finish_body(vals_ref, cols_ref, out_ref):
    vals = vals_ref[...]
    cols = cols_ref[...]
    rowi = lax.broadcasted_iota(jnp.int32, (LANES, B * T), 0)
    keep = (rowi >= 1) & (rowi <= N_NEG) & (cols != cols[0:1, :])
    negsum = jnp.sum(jnp.where(keep, jnp.exp(vals), 0.0), axis=0,
                     keepdims=True)
    pos = vals[0:1, :]
    lse = jnp.log(jnp.exp(pos) + negsum)
    out_ref[0, 0] = jnp.sum(lse - pos) * (1.0 / (B * T))


def _finish(vals, cols):
    return pl.pallas_call(
        _finish_body,
        out_shape=jax.ShapeDtypeStruct((1, 1), jnp.float32),
    )(vals, cols)


def kernel(pred_tokens, original_tokens):
    orig_t = jnp.transpose(original_tokens, (0, 2, 3, 1)).reshape(B, T, D)
    pred_z = jnp.transpose(pred_tokens, (0, 3, 2, 1)).reshape(B, T, D)

    ghat = _gram(orig_t, pred_z)

    neg_inds = jax.random.randint(
        jax.random.key(42), (B, T * N_NEG), 0, T - 1).astype(jnp.int32)

    t = jnp.arange(T, dtype=jnp.int32)
    zmap = (t % W) * H + t // W
    posrow = jnp.tile(zmap, (B,))
    negrows = jnp.transpose(
        neg_inds.reshape(B, T, N_NEG), (2, 0, 1)).reshape(N_NEG, B * T)
    cols = jnp.concatenate(
        [posrow[None], negrows,
         jnp.broadcast_to(posrow[None], (LANES - 1 - N_NEG, B * T))], axis=0)

    base = (jnp.repeat(jnp.arange(B, dtype=jnp.int32), T) * T
            + jnp.tile(t, (B,))) * T
    all_idx = (base[None, :] + cols).reshape(NW, NJ, 128)

    gathered = _sc_gather(ghat.reshape(B * T * T, 1), all_idx)
    vals = gathered.reshape(LANES, B * T)

    loss = _finish(vals, cols)
    return loss.reshape(())


# speedup vs baseline: 6.3712x; 6.3712x over previous
import functools

import jax
import jax.numpy as jnp
from jax import lax
from jax.experimental import pallas as pl
from jax.experimental.pallas import tpu as pltpu
from jax.experimental.pallas import tpu_sc as plsc

TEMPERATURE = 0.1
N_NEG = 10
EPS = 1e-8

B, D, H, W = 8, 512, 8, 128
T = H * W
R = B * T
LANES = 16
NW = 32
TOK_W = R // NW
CH = 32
NCH = TOK_W // CH


def _gram_body(o_ref, p_ref, out_ref):
    o = o_ref[0]
    p = p_ref[0]
    dots = lax.dot_general(o, p, (((1,), (1,)), ((), ())),
                           preferred_element_type=jnp.float32)
    no = jnp.maximum(jnp.sqrt(jnp.sum(o * o, axis=1, keepdims=True)), EPS)
    npv = jnp.maximum(jnp.sqrt(jnp.sum(p * p, axis=1)), EPS).reshape(1, T)
    out_ref[0] = dots * (1.0 / no) * (1.0 / npv) * (1.0 / TEMPERATURE)


def _gram(orig_t, pred_z):
    return pl.pallas_call(
        _gram_body,
        grid=(B,),
        in_specs=[
            pl.BlockSpec((1, T, D), lambda b: (b, 0, 0)),
            pl.BlockSpec((1, T, D), lambda b: (b, 0, 0)),
        ],
        out_specs=pl.BlockSpec((1, T, T), lambda b: (b, 0, 0)),
        out_shape=jax.ShapeDtypeStruct((B, T, T), jnp.float32),
    )(orig_t, pred_z)


@functools.partial(
    pl.kernel,
    mesh=plsc.VectorSubcoreMesh(core_axis_name="c", subcore_axis_name="s"),
    out_type=jax.ShapeDtypeStruct((NW, TOK_W * LANES), jnp.float32),
    compiler_params=pltpu.CompilerParams(
        use_tc_tiling_on_sc=False, needs_layout_passes=False),
    scratch_types=[
        pltpu.VMEM((TOK_W * LANES,), jnp.int32),
        pltpu.VMEM((CH, T), jnp.float32),
        pltpu.VMEM((TOK_W * LANES,), jnp.float32),
    ],
)
def _sc_gather(ghat_hbm, cols_hbm, out_hbm, idx_v, rows_v, out_v):
    wid = lax.axis_index("s") * 2 + lax.axis_index("c")
    base_tok = wid * TOK_W
    pltpu.sync_copy(cols_hbm.at[wid], idx_v)
    for c in range(NCH):
        pltpu.sync_copy(ghat_hbm.at[pl.ds(base_tok + c * CH, CH)], rows_v)

        def body(i, carry, c=c):
            row = jnp.full((LANES,), i, jnp.int32)
            off = (c * CH + i) * LANES
            col = idx_v[pl.ds(off, LANES)]
            out_v[pl.ds(off, LANES)] = plsc.load_gather(rows_v, [row, col])
            return carry

        lax.fori_loop(0, CH, body, 0)
    pltpu.sync_copy(out_v, out_hbm.at[wid])


def _finish_body(vals_ref, cols_ref, out_ref):
    vals = vals_ref[...]
    cols = cols_ref[...]
    lane = lax.broadcasted_iota(jnp.int32, (R, LANES), 1)
    keep = (lane >= 1) & (lane <= N_NEG) & (cols != cols[:, 0:1])
    negsum = jnp.sum(jnp.where(keep, jnp.exp(vals), 0.0), axis=1,
                     keepdims=True)
    pos = vals[:, 0:1]
    lse = jnp.log(jnp.exp(pos) + negsum)
    out_ref[...] = jnp.sum(lse - pos, keepdims=True) * (1.0 / R)


def _finish(vals, cols):
    return pl.pallas_call(
        _finish_body,
        out_shape=jax.ShapeDtypeStruct((1, 1), jnp.float32),
    )(vals, cols)


def kernel(pred_tokens, original_tokens):
    orig_t = jnp.transpose(original_tokens, (0, 2, 3, 1)).reshape(B, T, D)
    pred_z = jnp.transpose(pred_tokens, (0, 3, 2, 1)).reshape(B, T, D)

    ghat = _gram(orig_t, pred_z)

    neg_inds = jax.random.randint(
        jax.random.key(42), (B, T * N_NEG), 0, T - 1).astype(jnp.int32)

    t = jnp.arange(T, dtype=jnp.int32)
    zmap = (t % W) * H + t // W
    poscol = jnp.tile(zmap, (B,))[:, None]
    cols = jnp.concatenate(
        [poscol, neg_inds.reshape(R, N_NEG),
         jnp.broadcast_to(poscol, (R, LANES - 1 - N_NEG))], axis=1)

    gathered = _sc_gather(ghat.reshape(R, T), cols.reshape(NW, TOK_W * LANES))
    vals = gathered.reshape(R, LANES)

    loss = _finish(vals, cols)
    return loss.reshape(())
